# baseline (speedup 1.0000x reference)
import functools

import jax
import jax.numpy as jnp
from jax import lax
from jax.experimental import pallas as pl
from jax.experimental.pallas import tpu as pltpu
from jax.experimental.pallas import tpu_sc as plsc

NUM_EMBEDDINGS = 1000000
EMBEDDING_DIM = 64
BATCH = 16384
HIST = 50

_TOTAL = BATCH * HIST
_CHUNK = 128
_NBUF = 10


def _make_gather(num_workers: int, num_cores: int):
    chunks_per_w = _TOTAL // (num_workers * _CHUNK)
    per_w = chunks_per_w * _CHUNK
    outer = chunks_per_w // _NBUF
    mesh = plsc.VectorSubcoreMesh(core_axis_name="c", subcore_axis_name="s")

    @functools.partial(
        pl.kernel,
        mesh=mesh,
        out_type=jax.ShapeDtypeStruct((_TOTAL, EMBEDDING_DIM), jnp.float32),
        scratch_types=[
            pltpu.VMEM((chunks_per_w, _CHUNK), jnp.int32),
            pltpu.VMEM((_NBUF, _CHUNK, EMBEDDING_DIM), jnp.float32),
            [pltpu.SemaphoreType.DMA] * _NBUF,
        ],
        compiler_params=pltpu.CompilerParams(use_tc_tiling_on_sc=False),
    )
    def gather_kernel(idx_hbm, table_hbm, out_hbm, idx_v, rows_v, sems):
        wid = lax.axis_index("s") * num_cores + lax.axis_index("c")
        base = wid * per_w
        pltpu.sync_copy(idx_hbm.at[wid], idx_v)

        def fire(j, b):
            pltpu.async_copy(table_hbm.at[idx_v.at[j]], rows_v.at[b], sems[b])

        def drain(j, b):
            pltpu.make_async_copy(
                table_hbm.at[idx_v.at[j]], rows_v.at[b], sems[b]
            ).wait()

        for b in range(_NBUF):
            fire(b, b)

        def body(j2, carry):
            for b in range(_NBUF):
                j = j2 * _NBUF + b
                drain(j, b)
                pltpu.sync_copy(
                    rows_v.at[b],
                    out_hbm.at[pl.ds(base + j * _CHUNK, _CHUNK)],
                )

                @pl.when(j2 + 1 < outer)
                def _():
                    fire(j + _NBUF, b)

            return carry

        lax.fori_loop(0, outer, body, 0)

    return gather_kernel


def kernel(token_ids, embedding):
    info = plsc.get_sparse_core_info()
    num_workers = info.num_cores * info.num_subcores
    idx = token_ids.reshape(num_workers, -1, _CHUNK).astype(jnp.int32)
    out = _make_gather(num_workers, info.num_cores)(idx, embedding)
    return out.reshape(BATCH, HIST, EMBEDDING_DIM)
